# Initial kernel scaffold; baseline (speedup 1.0000x reference)
import jax
import jax.numpy as jnp
from jax.experimental import pallas as pl


def kernel(x, edge_index, edge_weight, W1, b1, W2, b2, Wl1, bl1, Wl2, bl2):
    raise NotImplementedError("write your pallas kernel here")


# speedup vs baseline: 9.0479x; 9.0479x over previous
import functools

import jax
import jax.numpy as jnp
from jax import lax
from jax.experimental import pallas as pl
from jax.experimental.pallas import tpu as pltpu
from jax.experimental.pallas import tpu_sc as plsc

NC = 2
NS = 16
CHUNK = 80
DEG_W = 16


def _zero_vmem(buf, n_rows, width):
    zero = jnp.zeros((16,), jnp.float32)
    for i in range(n_rows):
        for f in range(width // 16):
            buf[i, pl.ds(f * 16, 16)] = zero


ZR = 80


def _row_split(n_nodes):
    big = -(-n_nodes // NS)
    big = -(-big // ZR) * ZR
    last = n_nodes - (NS - 1) * big
    assert last >= 0 and last % ZR == 0
    return big, last


def _ranged_copy(s, n_nodes, do_copy):
    big, last = _row_split(n_nodes)

    @pl.when(s < NS - 1)
    def _():
        for i in range(big // ZR):
            do_copy(pl.multiple_of(s * big + i * ZR, 8), ZR)

    @pl.when(s == NS - 1)
    def _():
        for i in range(last // ZR):
            do_copy((NS - 1) * big + i * ZR, ZR)


@functools.partial(jax.jit, static_argnames=("n_nodes", "width"))
def _sc_deg(col, ew, n_nodes, width=128):
    E = col.shape[0]
    EP = E // (NC * NS)
    n_chunks = EP // CHUNK
    mesh = plsc.VectorSubcoreMesh(core_axis_name="c", subcore_axis_name="s")

    @functools.partial(
        pl.kernel, mesh=mesh,
        out_type=(jax.ShapeDtypeStruct((n_nodes, width), jnp.float32),
                  jax.ShapeDtypeStruct((n_nodes, width), jnp.float32)),
        scratch_types=[
            pltpu.VMEM((CHUNK,), jnp.int32),
            pltpu.VMEM((CHUNK,), jnp.float32),
            pltpu.VMEM((CHUNK, width), jnp.float32),
            pltpu.VMEM((ZR, width), jnp.float32),
            pltpu.VMEM_SHARED((n_nodes, width), jnp.float32),
        ],
    )
    def deg_kernel(col_hbm, ew_hbm, out0, out1, cidx, ewv, rows, zbuf, acc):
        c = lax.axis_index("c")
        s = lax.axis_index("s")
        _zero_vmem(zbuf, ZR, width)
        _ranged_copy(s, n_nodes,
                     lambda off, n: pltpu.sync_copy(zbuf, acc.at[pl.ds(off, n)]))
        plsc.subcore_barrier()

        def body(t, carry):
            base = (c * NS + s) * EP + t * CHUNK
            pltpu.sync_copy(col_hbm.at[pl.ds(base, CHUNK)], cidx)
            pltpu.sync_copy(ew_hbm.at[pl.ds(base, CHUNK)], ewv)
            for gi in range(CHUNK // 16):
                ewg = ewv[pl.ds(gi * 16, 16)]
                for j in range(16):
                    m = gi * 16 + j
                    w = jnp.full((16,), ewg[j], jnp.float32)
                    for f in range(width // 16):
                        rows[m, pl.ds(f * 16, 16)] = w
            pltpu.sync_copy(rows, acc.at[cidx], add=True)
            return carry

        lax.fori_loop(0, n_chunks, body, 0)
        plsc.subcore_barrier()

        @pl.when(c == 0)
        def _():
            _ranged_copy(s, n_nodes,
                         lambda off, n: pltpu.sync_copy(
                             acc.at[pl.ds(off, n)], out0.at[pl.ds(off, n)]))

        @pl.when(c == 1)
        def _():
            _ranged_copy(s, n_nodes,
                         lambda off, n: pltpu.sync_copy(
                             acc.at[pl.ds(off, n)], out1.at[pl.ds(off, n)]))

    return deg_kernel(col, ew)


@functools.partial(jax.jit, static_argnames=())
def _sc_agg(g, row, col, ew):
    n_nodes, D = g.shape
    E = row.shape[0]
    EP = E // (NC * NS)
    n_chunks = EP // CHUNK
    mesh = plsc.VectorSubcoreMesh(core_axis_name="c", subcore_axis_name="s")

    @functools.partial(
        pl.kernel, mesh=mesh,
        out_type=(jax.ShapeDtypeStruct((n_nodes, D), jnp.float32),
                  jax.ShapeDtypeStruct((n_nodes, D), jnp.float32)),
        scratch_types=[
            pltpu.VMEM((CHUNK,), jnp.int32),
            pltpu.VMEM((CHUNK,), jnp.int32),
            pltpu.VMEM((CHUNK,), jnp.float32),
            pltpu.VMEM((CHUNK, D), jnp.float32),
            pltpu.VMEM((ZR, D), jnp.float32),
            pltpu.VMEM_SHARED((n_nodes, D), jnp.float32),
            pltpu.SemaphoreType.DMA,
        ],
    )
    def agg_kernel(g_hbm, row_hbm, col_hbm, ew_hbm, out0, out1,
                   ridx, cidx, ewv, rows, zbuf, acc, sem):
        c = lax.axis_index("c")
        s = lax.axis_index("s")
        _zero_vmem(zbuf, ZR, D)
        _ranged_copy(s, n_nodes,
                     lambda off, n: pltpu.sync_copy(zbuf, acc.at[pl.ds(off, n)]))
        plsc.subcore_barrier()

        def body(t, carry):
            base = (c * NS + s) * EP + t * CHUNK
            pltpu.sync_copy(row_hbm.at[pl.ds(base, CHUNK)], ridx)
            pltpu.sync_copy(col_hbm.at[pl.ds(base, CHUNK)], cidx)
            pltpu.sync_copy(ew_hbm.at[pl.ds(base, CHUNK)], ewv)
            pltpu.async_copy(g_hbm.at[ridx], rows, sem).wait()
            for gi in range(CHUNK // 16):
                ewg = ewv[pl.ds(gi * 16, 16)]
                for j in range(16):
                    m = gi * 16 + j
                    w = jnp.full((16,), ewg[j], jnp.float32)
                    for f in range(D // 16):
                        sl = pl.ds(f * 16, 16)
                        rows[m, sl] = rows[m, sl] * w
            pltpu.sync_copy(rows, acc.at[cidx], add=True)
            return carry

        lax.fori_loop(0, n_chunks, body, 0)
        plsc.subcore_barrier()

        @pl.when(c == 0)
        def _():
            _ranged_copy(s, n_nodes,
                         lambda off, n: pltpu.sync_copy(
                             acc.at[pl.ds(off, n)], out0.at[pl.ds(off, n)]))

        @pl.when(c == 1)
        def _():
            _ranged_copy(s, n_nodes,
                         lambda off, n: pltpu.sync_copy(
                             acc.at[pl.ds(off, n)], out1.at[pl.ds(off, n)]))

    return agg_kernel(g, row, col, ew)


_BLK = 1000


def _tc_stage0(deg0, deg1, x, W1):
    N, DF = x.shape
    DH = W1.shape[1]
    DW = deg0.shape[1]

    def body(d0, d1, xr, wr, disr, gr):
        deg = d0[...][:, 0] + d1[...][:, 0] + 1.0
        dis = jnp.where(deg > 0, lax.rsqrt(deg), 0.0)[:, None]
        disr[...] = dis
        gr[...] = jnp.dot(xr[...], wr[...],
                          preferred_element_type=jnp.float32) * dis

    return pl.pallas_call(
        body,
        grid=(N // _BLK,),
        in_specs=[pl.BlockSpec((_BLK, DW), lambda i: (i, 0)),
                  pl.BlockSpec((_BLK, DW), lambda i: (i, 0)),
                  pl.BlockSpec((_BLK, DF), lambda i: (i, 0)),
                  pl.BlockSpec((DF, DH), lambda i: (0, 0))],
        out_specs=[pl.BlockSpec((_BLK, 1), lambda i: (i, 0)),
                   pl.BlockSpec((_BLK, DH), lambda i: (i, 0))],
        out_shape=[jax.ShapeDtypeStruct((N, 1), jnp.float32),
                   jax.ShapeDtypeStruct((N, DH), jnp.float32)],
    )(deg0, deg1, x, W1)


def _tc_stage1(a0, a1, g1, dis, b1, W2):
    N, DH = g1.shape

    def body(a0r, a1r, gr, disr, br, wr, outr):
        dis_ = disr[...]
        out1 = (a0r[...] + a1r[...] + gr[...]) * dis_ + br[...]
        outr[...] = jnp.dot(out1, wr[...],
                            preferred_element_type=jnp.float32) * dis_

    return pl.pallas_call(
        body,
        grid=(N // _BLK,),
        in_specs=[pl.BlockSpec((_BLK, DH), lambda i: (i, 0)),
                  pl.BlockSpec((_BLK, DH), lambda i: (i, 0)),
                  pl.BlockSpec((_BLK, DH), lambda i: (i, 0)),
                  pl.BlockSpec((_BLK, 1), lambda i: (i, 0)),
                  pl.BlockSpec((1, DH), lambda i: (0, 0)),
                  pl.BlockSpec((DH, DH), lambda i: (0, 0))],
        out_specs=pl.BlockSpec((_BLK, DH), lambda i: (i, 0)),
        out_shape=jax.ShapeDtypeStruct((N, DH), jnp.float32),
    )(a0, a1, g1, dis, b1, W2)


def _tc_stage2(a0, a1, g2, dis, b2, Wl1, bl1, Wl2, bl2):
    N, DH = g2.shape
    DO = Wl2.shape[1]

    def body(a0r, a1r, gr, disr, br, w1r, b1r, w2r, b2r, yr):
        dis_ = disr[...]
        out2 = (a0r[...] + a1r[...] + gr[...]) * dis_ + br[...]
        m = jnp.dot(out2, w1r[...], preferred_element_type=jnp.float32)
        m = jnp.maximum(m + b1r[...], 0.0)
        yr[...] = jnp.dot(m, w2r[...],
                          preferred_element_type=jnp.float32) + b2r[...]

    return pl.pallas_call(
        body,
        grid=(N // _BLK,),
        in_specs=[pl.BlockSpec((_BLK, DH), lambda i: (i, 0)),
                  pl.BlockSpec((_BLK, DH), lambda i: (i, 0)),
                  pl.BlockSpec((_BLK, DH), lambda i: (i, 0)),
                  pl.BlockSpec((_BLK, 1), lambda i: (i, 0)),
                  pl.BlockSpec((1, DH), lambda i: (0, 0)),
                  pl.BlockSpec((DH, DH), lambda i: (0, 0)),
                  pl.BlockSpec((1, DH), lambda i: (0, 0)),
                  pl.BlockSpec((DH, DO), lambda i: (0, 0)),
                  pl.BlockSpec((1, DO), lambda i: (0, 0))],
        out_specs=pl.BlockSpec((_BLK, DO), lambda i: (i, 0)),
        out_shape=jax.ShapeDtypeStruct((N, DO), jnp.float32),
    )(a0, a1, g2, dis, b2, Wl1, bl1, Wl2, bl2)


def kernel(x, edge_index, edge_weight, W1, b1, W2, b2, Wl1, bl1, Wl2, bl2):
    n_nodes = x.shape[0]
    row = edge_index[0]
    col = edge_index[1]
    ew = edge_weight.astype(jnp.float32)

    deg0, deg1 = _sc_deg(col, ew, n_nodes)
    dis, g1 = _tc_stage0(deg0, deg1, x, W1)
    a0, a1 = _sc_agg(g1, row, col, ew)
    g2 = _tc_stage1(a0, a1, g1, dis, b1.reshape(1, -1), W2)
    c0, c1 = _sc_agg(g2, row, col, ew)
    y = _tc_stage2(c0, c1, g2, dis, b2.reshape(1, -1),
                   Wl1, bl1.reshape(1, -1), Wl2, bl2.reshape(1, -1))
    return y


# speedup vs baseline: 21.3386x; 2.3584x over previous
import functools

import jax
import jax.numpy as jnp
from jax import lax
from jax.experimental import pallas as pl
from jax.experimental.pallas import tpu as pltpu
from jax.experimental.pallas import tpu_sc as plsc

NC = 2
NS = 16
CHUNK = 80
DEG_W = 16


def _zero_vmem(buf, n_rows, width):
    zero = jnp.zeros((16,), jnp.float32)
    for i in range(n_rows):
        for f in range(width // 16):
            buf[i, pl.ds(f * 16, 16)] = zero


ZR = 80


def _row_split(n_nodes):
    big = -(-n_nodes // NS)
    big = -(-big // ZR) * ZR
    last = n_nodes - (NS - 1) * big
    assert last >= 0 and last % ZR == 0
    return big, last


def _ranged_copy(s, n_nodes, do_copy):
    big, last = _row_split(n_nodes)

    @pl.when(s < NS - 1)
    def _():
        for i in range(big // ZR):
            do_copy(pl.multiple_of(s * big + i * ZR, 8), ZR)

    @pl.when(s == NS - 1)
    def _():
        for i in range(last // ZR):
            do_copy((NS - 1) * big + i * ZR, ZR)


@functools.partial(jax.jit, static_argnames=("n_nodes", "width"))
def _sc_deg(col, ew, n_nodes, width=128):
    E = col.shape[0]
    EP = E // (NC * NS)
    n_chunks = EP // CHUNK
    mesh = plsc.VectorSubcoreMesh(core_axis_name="c", subcore_axis_name="s")

    @functools.partial(
        pl.kernel, mesh=mesh,
        out_type=(jax.ShapeDtypeStruct((n_nodes, width), jnp.float32),
                  jax.ShapeDtypeStruct((n_nodes, width), jnp.float32)),
        scratch_types=[
            pltpu.VMEM((EP,), jnp.float32),
            pltpu.VMEM((2, CHUNK), jnp.int32),
            pltpu.VMEM((2, CHUNK, width), jnp.float32),
            pltpu.VMEM_SHARED((n_nodes, width), jnp.float32),
            pltpu.SemaphoreType.DMA,
            pltpu.SemaphoreType.DMA,
            pltpu.SemaphoreType.DMA,
            pltpu.SemaphoreType.DMA,
        ],
    )
    def deg_kernel(col_hbm, ew_hbm, out0, out1, ew_all, cidx, rows, acc,
                   ss0, ss1, is0, is1):
        ssem = [ss0, ss1]
        isem = [is0, is1]
        c = lax.axis_index("c")
        s = lax.axis_index("s")
        tb = (c * NS + s) * EP

        zero16 = jnp.zeros((16,), jnp.float32)
        for i in range(ZR):
            for f in range(width // 16):
                rows[0, i, pl.ds(f * 16, 16)] = zero16
        _ranged_copy(s, n_nodes,
                     lambda off, n: pltpu.sync_copy(
                         rows.at[0], acc.at[pl.ds(off, n)]))
        plsc.subcore_barrier()

        pltpu.sync_copy(ew_hbm.at[pl.ds(tb, EP)], ew_all)

        def fill_cidx(t, buf):
            off = pl.multiple_of(t * CHUNK, 8)
            pltpu.async_copy(col_hbm.at[pl.ds(tb + off, CHUNK)],
                             cidx.at[buf], isem[buf])

        def drain_scatter(buf):
            pltpu.make_async_copy(rows.at[buf], acc.at[cidx.at[buf]],
                                  ssem[buf]).wait()

        def handle(t, buf, other):
            @pl.when(t >= 1)
            def _():
                drain_scatter(other)

            @pl.when(t + 1 < n_chunks)
            def _():
                fill_cidx(t + 1, other)

            def fill_body(gi, carry):
                goff = pl.multiple_of(t * CHUNK + gi * 16, 8)
                ewg = ew_all[pl.ds(goff, 16)]
                for j in range(16):
                    m = gi * 16 + j
                    w = jnp.full((16,), ewg[j], jnp.float32)
                    for f in range(width // 16):
                        rows[buf, m, pl.ds(f * 16, 16)] = w
                return carry

            lax.fori_loop(0, CHUNK // 16, fill_body, 0)

            pltpu.make_async_copy(col_hbm.at[pl.ds(tb, CHUNK)],
                                  cidx.at[buf], isem[buf]).wait()
            pltpu.async_copy(rows.at[buf], acc.at[cidx.at[buf]],
                             ssem[buf], add=True)

        fill_cidx(0, 0)

        def pair(i, carry):
            handle(2 * i, 0, 1)
            handle(2 * i + 1, 1, 0)
            return carry

        lax.fori_loop(0, n_chunks // 2, pair, 0)
        if n_chunks % 2:
            handle(n_chunks - 1, 1 - (n_chunks % 2), n_chunks % 2)
        drain_scatter((n_chunks - 1) % 2)
        plsc.subcore_barrier()

        @pl.when(c == 0)
        def _():
            _ranged_copy(s, n_nodes,
                         lambda off, n: pltpu.sync_copy(
                             acc.at[pl.ds(off, n)], out0.at[pl.ds(off, n)]))

        @pl.when(c == 1)
        def _():
            _ranged_copy(s, n_nodes,
                         lambda off, n: pltpu.sync_copy(
                             acc.at[pl.ds(off, n)], out1.at[pl.ds(off, n)]))

    return deg_kernel(col, ew)


@functools.partial(jax.jit, static_argnames=())
def _sc_agg(g, row, col, ew):
    n_nodes, D = g.shape
    E = row.shape[0]
    EP = E // (NC * NS)
    n_chunks = EP // CHUNK
    mesh = plsc.VectorSubcoreMesh(core_axis_name="c", subcore_axis_name="s")

    @functools.partial(
        pl.kernel, mesh=mesh,
        out_type=(jax.ShapeDtypeStruct((n_nodes, D), jnp.float32),
                  jax.ShapeDtypeStruct((n_nodes, D), jnp.float32)),
        scratch_types=[
            pltpu.VMEM((EP,), jnp.int32),
            pltpu.VMEM((EP,), jnp.float32),
            pltpu.VMEM((2, CHUNK), jnp.int32),
            pltpu.VMEM((2, CHUNK, D), jnp.float32),
            pltpu.VMEM_SHARED((n_nodes, D), jnp.float32),
            pltpu.SemaphoreType.DMA,
            pltpu.SemaphoreType.DMA,
            pltpu.SemaphoreType.DMA,
            pltpu.SemaphoreType.DMA,
            pltpu.SemaphoreType.DMA,
            pltpu.SemaphoreType.DMA,
        ],
    )
    def agg_kernel(g_hbm, row_hbm, col_hbm, ew_hbm, out0, out1,
                   ridx_all, ew_all, cidx, rows, acc,
                   gs0, gs1, ss0, ss1, is0, is1):
        gsem = [gs0, gs1]
        ssem = [ss0, ss1]
        isem = [is0, is1]
        c = lax.axis_index("c")
        s = lax.axis_index("s")
        tb = (c * NS + s) * EP

        zero16 = jnp.zeros((16,), jnp.float32)
        for i in range(ZR):
            for f in range(D // 16):
                rows[0, i, pl.ds(f * 16, 16)] = zero16
        _ranged_copy(s, n_nodes,
                     lambda off, n: pltpu.sync_copy(
                         rows.at[0], acc.at[pl.ds(off, n)]))
        plsc.subcore_barrier()

        pltpu.sync_copy(row_hbm.at[pl.ds(tb, EP)], ridx_all)
        pltpu.sync_copy(ew_hbm.at[pl.ds(tb, EP)], ew_all)

        def start_chunk(t, buf):
            off = pl.multiple_of(t * CHUNK, 8)
            pltpu.async_copy(col_hbm.at[pl.ds(tb + off, CHUNK)],
                             cidx.at[buf], isem[buf])
            pltpu.async_copy(g_hbm.at[ridx_all.at[pl.ds(off, CHUNK)]],
                             rows.at[buf], gsem[buf])

        def drain_scatter(buf):
            pltpu.make_async_copy(rows.at[buf], acc.at[cidx.at[buf]],
                                  ssem[buf]).wait()

        def handle(t, buf, other):
            @pl.when(t >= 1)
            def _():
                drain_scatter(other)

            @pl.when(t + 1 < n_chunks)
            def _():
                start_chunk(t + 1, other)

            pltpu.make_async_copy(g_hbm.at[ridx_all.at[pl.ds(0, CHUNK)]],
                                  rows.at[buf], gsem[buf]).wait()

            def scale_body(gi, carry):
                goff = pl.multiple_of(t * CHUNK + gi * 16, 8)
                ewg = ew_all[pl.ds(goff, 16)]
                for j in range(16):
                    m = gi * 16 + j
                    w = jnp.full((16,), ewg[j], jnp.float32)
                    for f in range(D // 16):
                        sl = pl.ds(f * 16, 16)
                        rows[buf, m, sl] = rows[buf, m, sl] * w
                return carry

            lax.fori_loop(0, CHUNK // 16, scale_body, 0)

            pltpu.make_async_copy(col_hbm.at[pl.ds(tb, CHUNK)],
                                  cidx.at[buf], isem[buf]).wait()
            pltpu.async_copy(rows.at[buf], acc.at[cidx.at[buf]],
                             ssem[buf], add=True)

        start_chunk(0, 0)

        def pair(i, carry):
            handle(2 * i, 0, 1)
            handle(2 * i + 1, 1, 0)
            return carry

        lax.fori_loop(0, n_chunks // 2, pair, 0)
        if n_chunks % 2:
            handle(n_chunks - 1, 1 - (n_chunks % 2), n_chunks % 2)
        drain_scatter((n_chunks - 1) % 2)
        plsc.subcore_barrier()

        @pl.when(c == 0)
        def _():
            _ranged_copy(s, n_nodes,
                         lambda off, n: pltpu.sync_copy(
                             acc.at[pl.ds(off, n)], out0.at[pl.ds(off, n)]))

        @pl.when(c == 1)
        def _():
            _ranged_copy(s, n_nodes,
                         lambda off, n: pltpu.sync_copy(
                             acc.at[pl.ds(off, n)], out1.at[pl.ds(off, n)]))

    return agg_kernel(g, row, col, ew)


_BLK = 1000


def _tc_stage0(deg0, deg1, x, W1):
    N, DF = x.shape
    DH = W1.shape[1]
    DW = deg0.shape[1]

    def body(d0, d1, xr, wr, disr, gr):
        deg = d0[...][:, 0] + d1[...][:, 0] + 1.0
        dis = jnp.where(deg > 0, lax.rsqrt(deg), 0.0)[:, None]
        disr[...] = dis
        gr[...] = jnp.dot(xr[...], wr[...],
                          preferred_element_type=jnp.float32) * dis

    return pl.pallas_call(
        body,
        grid=(N // _BLK,),
        in_specs=[pl.BlockSpec((_BLK, DW), lambda i: (i, 0)),
                  pl.BlockSpec((_BLK, DW), lambda i: (i, 0)),
                  pl.BlockSpec((_BLK, DF), lambda i: (i, 0)),
                  pl.BlockSpec((DF, DH), lambda i: (0, 0))],
        out_specs=[pl.BlockSpec((_BLK, 1), lambda i: (i, 0)),
                   pl.BlockSpec((_BLK, DH), lambda i: (i, 0))],
        out_shape=[jax.ShapeDtypeStruct((N, 1), jnp.float32),
                   jax.ShapeDtypeStruct((N, DH), jnp.float32)],
    )(deg0, deg1, x, W1)


def _tc_stage1(a0, a1, g1, dis, b1, W2):
    N, DH = g1.shape

    def body(a0r, a1r, gr, disr, br, wr, outr):
        dis_ = disr[...]
        out1 = (a0r[...] + a1r[...] + gr[...]) * dis_ + br[...]
        outr[...] = jnp.dot(out1, wr[...],
                            preferred_element_type=jnp.float32) * dis_

    return pl.pallas_call(
        body,
        grid=(N // _BLK,),
        in_specs=[pl.BlockSpec((_BLK, DH), lambda i: (i, 0)),
                  pl.BlockSpec((_BLK, DH), lambda i: (i, 0)),
                  pl.BlockSpec((_BLK, DH), lambda i: (i, 0)),
                  pl.BlockSpec((_BLK, 1), lambda i: (i, 0)),
                  pl.BlockSpec((1, DH), lambda i: (0, 0)),
                  pl.BlockSpec((DH, DH), lambda i: (0, 0))],
        out_specs=pl.BlockSpec((_BLK, DH), lambda i: (i, 0)),
        out_shape=jax.ShapeDtypeStruct((N, DH), jnp.float32),
    )(a0, a1, g1, dis, b1, W2)


def _tc_stage2(a0, a1, g2, dis, b2, Wl1, bl1, Wl2, bl2):
    N, DH = g2.shape
    DO = Wl2.shape[1]

    def body(a0r, a1r, gr, disr, br, w1r, b1r, w2r, b2r, yr):
        dis_ = disr[...]
        out2 = (a0r[...] + a1r[...] + gr[...]) * dis_ + br[...]
        m = jnp.dot(out2, w1r[...], preferred_element_type=jnp.float32)
        m = jnp.maximum(m + b1r[...], 0.0)
        yr[...] = jnp.dot(m, w2r[...],
                          preferred_element_type=jnp.float32) + b2r[...]

    return pl.pallas_call(
        body,
        grid=(N // _BLK,),
        in_specs=[pl.BlockSpec((_BLK, DH), lambda i: (i, 0)),
                  pl.BlockSpec((_BLK, DH), lambda i: (i, 0)),
                  pl.BlockSpec((_BLK, DH), lambda i: (i, 0)),
                  pl.BlockSpec((_BLK, 1), lambda i: (i, 0)),
                  pl.BlockSpec((1, DH), lambda i: (0, 0)),
                  pl.BlockSpec((DH, DH), lambda i: (0, 0)),
                  pl.BlockSpec((1, DH), lambda i: (0, 0)),
                  pl.BlockSpec((DH, DO), lambda i: (0, 0)),
                  pl.BlockSpec((1, DO), lambda i: (0, 0))],
        out_specs=pl.BlockSpec((_BLK, DO), lambda i: (i, 0)),
        out_shape=jax.ShapeDtypeStruct((N, DO), jnp.float32),
    )(a0, a1, g2, dis, b2, Wl1, bl1, Wl2, bl2)


def kernel(x, edge_index, edge_weight, W1, b1, W2, b2, Wl1, bl1, Wl2, bl2):
    n_nodes = x.shape[0]
    row = edge_index[0]
    col = edge_index[1]
    ew = edge_weight.astype(jnp.float32)

    deg0, deg1 = _sc_deg(col, ew, n_nodes)
    dis, g1 = _tc_stage0(deg0, deg1, x, W1)
    a0, a1 = _sc_agg(g1, row, col, ew)
    g2 = _tc_stage1(a0, a1, g1, dis, b1.reshape(1, -1), W2)
    c0, c1 = _sc_agg(g2, row, col, ew)
    y = _tc_stage2(c0, c1, g2, dis, b2.reshape(1, -1),
                   Wl1, bl1.reshape(1, -1), Wl2, bl2.reshape(1, -1))
    return y


# speedup vs baseline: 27.1575x; 1.2727x over previous
import functools

import jax
import jax.numpy as jnp
from jax import lax
from jax.experimental import pallas as pl
from jax.experimental.pallas import tpu as pltpu
from jax.experimental.pallas import tpu_sc as plsc

NC = 2
NS = 16
CHUNK = 80
DEG_W = 16


def _zero_vmem(buf, n_rows, width):
    zero = jnp.zeros((16,), jnp.float32)
    for i in range(n_rows):
        for f in range(width // 16):
            buf[i, pl.ds(f * 16, 16)] = zero


ZR = 80


def _row_split(n_nodes):
    big = -(-n_nodes // NS)
    big = -(-big // ZR) * ZR
    last = n_nodes - (NS - 1) * big
    assert last >= 0 and last % ZR == 0
    return big, last


def _ranged_copy(s, n_nodes, do_copy):
    big, last = _row_split(n_nodes)

    @pl.when(s < NS - 1)
    def _():
        for i in range(big // ZR):
            do_copy(pl.multiple_of(s * big + i * ZR, 8), ZR)

    @pl.when(s == NS - 1)
    def _():
        for i in range(last // ZR):
            do_copy((NS - 1) * big + i * ZR, ZR)


@functools.partial(jax.jit, static_argnames=("n_nodes",))
def _sc_deg(col, ew, n_nodes):
    E = col.shape[0]
    NW = NC * NS
    EP = E // NW
    mesh = plsc.VectorSubcoreMesh(core_axis_name="c", subcore_axis_name="s")

    @functools.partial(
        pl.kernel, mesh=mesh,
        out_type=jax.ShapeDtypeStruct((NW * n_nodes,), jnp.float32),
        compiler_params=pltpu.CompilerParams(needs_layout_passes=False),
        scratch_types=[
            pltpu.VMEM((EP,), jnp.int32),
            pltpu.VMEM((EP,), jnp.float32),
            pltpu.VMEM((n_nodes,), jnp.float32),
        ],
    )
    def deg_kernel(col_hbm, ew_hbm, out, cidx_all, ew_all, acc):
        c = lax.axis_index("c")
        s = lax.axis_index("s")
        wid = c * NS + s
        tb = wid * EP

        zero16 = jnp.zeros((16,), jnp.float32)

        def zbody(i, carry):
            acc[pl.ds(pl.multiple_of(i * 16, 8), 16)] = zero16
            return carry

        lax.fori_loop(0, n_nodes // 16, zbody, 0)

        pltpu.sync_copy(col_hbm.at[pl.ds(tb, EP)], cidx_all)
        pltpu.sync_copy(ew_hbm.at[pl.ds(tb, EP)], ew_all)

        def body(t, carry):
            off = pl.multiple_of(t * 16, 8)
            cv = cidx_all[pl.ds(off, 16)]
            wv = ew_all[pl.ds(off, 16)]
            plsc.addupdate_scatter(acc, [cv], wv)
            return carry

        lax.fori_loop(0, EP // 16, body, 0)
        pltpu.sync_copy(acc,
                        out.at[pl.ds(pl.multiple_of(wid * n_nodes, 8),
                                     n_nodes)])

    return deg_kernel(col, ew)


@functools.partial(jax.jit, static_argnames=())
def _sc_agg(g, row, col, ew):
    n_nodes, D = g.shape
    E = row.shape[0]
    EP = E // (NC * NS)
    n_chunks = EP // CHUNK
    mesh = plsc.VectorSubcoreMesh(core_axis_name="c", subcore_axis_name="s")

    @functools.partial(
        pl.kernel, mesh=mesh,
        out_type=(jax.ShapeDtypeStruct((n_nodes, D), jnp.float32),
                  jax.ShapeDtypeStruct((n_nodes, D), jnp.float32)),
        scratch_types=[
            pltpu.VMEM((EP,), jnp.int32),
            pltpu.VMEM((EP,), jnp.float32),
            pltpu.VMEM((2, CHUNK), jnp.int32),
            pltpu.VMEM((2, CHUNK, D), jnp.float32),
            pltpu.VMEM_SHARED((n_nodes, D), jnp.float32),
            pltpu.SemaphoreType.DMA,
            pltpu.SemaphoreType.DMA,
            pltpu.SemaphoreType.DMA,
            pltpu.SemaphoreType.DMA,
            pltpu.SemaphoreType.DMA,
            pltpu.SemaphoreType.DMA,
        ],
    )
    def agg_kernel(g_hbm, row_hbm, col_hbm, ew_hbm, out0, out1,
                   ridx_all, ew_all, cidx, rows, acc,
                   gs0, gs1, ss0, ss1, is0, is1):
        gsem = [gs0, gs1]
        ssem = [ss0, ss1]
        isem = [is0, is1]
        c = lax.axis_index("c")
        s = lax.axis_index("s")
        tb = (c * NS + s) * EP

        zero16 = jnp.zeros((16,), jnp.float32)
        for i in range(ZR):
            for f in range(D // 16):
                rows[0, i, pl.ds(f * 16, 16)] = zero16
        _ranged_copy(s, n_nodes,
                     lambda off, n: pltpu.sync_copy(
                         rows.at[0], acc.at[pl.ds(off, n)]))
        plsc.subcore_barrier()

        pltpu.sync_copy(row_hbm.at[pl.ds(tb, EP)], ridx_all)
        pltpu.sync_copy(ew_hbm.at[pl.ds(tb, EP)], ew_all)

        def start_chunk(t, buf):
            off = pl.multiple_of(t * CHUNK, 8)
            pltpu.async_copy(col_hbm.at[pl.ds(tb + off, CHUNK)],
                             cidx.at[buf], isem[buf])
            pltpu.async_copy(g_hbm.at[ridx_all.at[pl.ds(off, CHUNK)]],
                             rows.at[buf], gsem[buf])

        def drain_scatter(buf):
            pltpu.make_async_copy(rows.at[buf], acc.at[cidx.at[buf]],
                                  ssem[buf]).wait()

        def handle(t, buf, other):
            @pl.when(t >= 1)
            def _():
                drain_scatter(other)

            @pl.when(t + 1 < n_chunks)
            def _():
                start_chunk(t + 1, other)

            pltpu.make_async_copy(g_hbm.at[ridx_all.at[pl.ds(0, CHUNK)]],
                                  rows.at[buf], gsem[buf]).wait()

            def scale_body(gi, carry):
                goff = pl.multiple_of(t * CHUNK + gi * 16, 8)
                ewg = ew_all[pl.ds(goff, 16)]
                for j in range(16):
                    m = gi * 16 + j
                    w = jnp.full((16,), ewg[j], jnp.float32)
                    for f in range(D // 16):
                        sl = pl.ds(f * 16, 16)
                        rows[buf, m, sl] = rows[buf, m, sl] * w
                return carry

            lax.fori_loop(0, CHUNK // 16, scale_body, 0)

            pltpu.make_async_copy(col_hbm.at[pl.ds(tb, CHUNK)],
                                  cidx.at[buf], isem[buf]).wait()
            pltpu.async_copy(rows.at[buf], acc.at[cidx.at[buf]],
                             ssem[buf], add=True)

        start_chunk(0, 0)

        def pair(i, carry):
            handle(2 * i, 0, 1)
            handle(2 * i + 1, 1, 0)
            return carry

        lax.fori_loop(0, n_chunks // 2, pair, 0)
        if n_chunks % 2:
            handle(n_chunks - 1, 1 - (n_chunks % 2), n_chunks % 2)
        drain_scatter((n_chunks - 1) % 2)
        plsc.subcore_barrier()

        @pl.when(c == 0)
        def _():
            _ranged_copy(s, n_nodes,
                         lambda off, n: pltpu.sync_copy(
                             acc.at[pl.ds(off, n)], out0.at[pl.ds(off, n)]))

        @pl.when(c == 1)
        def _():
            _ranged_copy(s, n_nodes,
                         lambda off, n: pltpu.sync_copy(
                             acc.at[pl.ds(off, n)], out1.at[pl.ds(off, n)]))

    return agg_kernel(g, row, col, ew)


_BLK = 1000


def _tc_dis(deg_parts):
    NW, N = deg_parts.shape

    def body(dp, disr):
        deg = jnp.sum(dp[...], axis=0) + 1.0
        disr[...] = jnp.where(deg > 0, lax.rsqrt(deg), 0.0)[:, None]

    return pl.pallas_call(
        body,
        grid=(1,),
        in_specs=[pl.BlockSpec((NW, N), lambda i: (0, 0))],
        out_specs=pl.BlockSpec((N, 1), lambda i: (0, 0)),
        out_shape=jax.ShapeDtypeStruct((N, 1), jnp.float32),
    )(deg_parts)


def _tc_stage0(dis, x, W1):
    N, DF = x.shape
    DH = W1.shape[1]

    def body(disr, xr, wr, gr):
        gr[...] = jnp.dot(xr[...], wr[...],
                          preferred_element_type=jnp.float32) * disr[...]

    return pl.pallas_call(
        body,
        grid=(N // _BLK,),
        in_specs=[pl.BlockSpec((_BLK, 1), lambda i: (i, 0)),
                  pl.BlockSpec((_BLK, DF), lambda i: (i, 0)),
                  pl.BlockSpec((DF, DH), lambda i: (0, 0))],
        out_specs=pl.BlockSpec((_BLK, DH), lambda i: (i, 0)),
        out_shape=jax.ShapeDtypeStruct((N, DH), jnp.float32),
    )(dis, x, W1)


def _tc_stage1(a0, a1, g1, dis, b1, W2):
    N, DH = g1.shape

    def body(a0r, a1r, gr, disr, br, wr, outr):
        dis_ = disr[...]
        out1 = (a0r[...] + a1r[...] + gr[...]) * dis_ + br[...]
        outr[...] = jnp.dot(out1, wr[...],
                            preferred_element_type=jnp.float32) * dis_

    return pl.pallas_call(
        body,
        grid=(N // _BLK,),
        in_specs=[pl.BlockSpec((_BLK, DH), lambda i: (i, 0)),
                  pl.BlockSpec((_BLK, DH), lambda i: (i, 0)),
                  pl.BlockSpec((_BLK, DH), lambda i: (i, 0)),
                  pl.BlockSpec((_BLK, 1), lambda i: (i, 0)),
                  pl.BlockSpec((1, DH), lambda i: (0, 0)),
                  pl.BlockSpec((DH, DH), lambda i: (0, 0))],
        out_specs=pl.BlockSpec((_BLK, DH), lambda i: (i, 0)),
        out_shape=jax.ShapeDtypeStruct((N, DH), jnp.float32),
    )(a0, a1, g1, dis, b1, W2)


def _tc_stage2(a0, a1, g2, dis, b2, Wl1, bl1, Wl2, bl2):
    N, DH = g2.shape
    DO = Wl2.shape[1]

    def body(a0r, a1r, gr, disr, br, w1r, b1r, w2r, b2r, yr):
        dis_ = disr[...]
        out2 = (a0r[...] + a1r[...] + gr[...]) * dis_ + br[...]
        m = jnp.dot(out2, w1r[...], preferred_element_type=jnp.float32)
        m = jnp.maximum(m + b1r[...], 0.0)
        yr[...] = jnp.dot(m, w2r[...],
                          preferred_element_type=jnp.float32) + b2r[...]

    return pl.pallas_call(
        body,
        grid=(N // _BLK,),
        in_specs=[pl.BlockSpec((_BLK, DH), lambda i: (i, 0)),
                  pl.BlockSpec((_BLK, DH), lambda i: (i, 0)),
                  pl.BlockSpec((_BLK, DH), lambda i: (i, 0)),
                  pl.BlockSpec((_BLK, 1), lambda i: (i, 0)),
                  pl.BlockSpec((1, DH), lambda i: (0, 0)),
                  pl.BlockSpec((DH, DH), lambda i: (0, 0)),
                  pl.BlockSpec((1, DH), lambda i: (0, 0)),
                  pl.BlockSpec((DH, DO), lambda i: (0, 0)),
                  pl.BlockSpec((1, DO), lambda i: (0, 0))],
        out_specs=pl.BlockSpec((_BLK, DO), lambda i: (i, 0)),
        out_shape=jax.ShapeDtypeStruct((N, DO), jnp.float32),
    )(a0, a1, g2, dis, b2, Wl1, bl1, Wl2, bl2)


def kernel(x, edge_index, edge_weight, W1, b1, W2, b2, Wl1, bl1, Wl2, bl2):
    n_nodes = x.shape[0]
    row = edge_index[0]
    col = edge_index[1]
    ew = edge_weight.astype(jnp.float32)

    deg_flat = _sc_deg(col, ew, n_nodes)
    dis = _tc_dis(deg_flat.reshape(NC * NS, n_nodes))
    g1 = _tc_stage0(dis, x, W1)
    a0, a1 = _sc_agg(g1, row, col, ew)
    g2 = _tc_stage1(a0, a1, g1, dis, b1.reshape(1, -1), W2)
    c0, c1 = _sc_agg(g2, row, col, ew)
    y = _tc_stage2(c0, c1, g2, dis, b2.reshape(1, -1),
                   Wl1, bl1.reshape(1, -1), Wl2, bl2.reshape(1, -1))
    return y
